# baseline (speedup 1.0000x reference)
import functools

import jax
import jax.numpy as jnp
from jax.experimental import pallas as pl
from jax.experimental.pallas import tpu as pltpu

T = 2048
D = 1024
F = 2048
E = 8
TOPK = 2
TK = T * TOPK
BT = 128
NPAD = TK + E * BT
NB = NPAD // BT

_NEG = -1e30


def _router_body(x_ref, rw_ref, logits_ref, i1_ref, i2_ref, w1_ref, w2_ref):
    x = x_ref[...]
    rw = rw_ref[...]
    logits = jax.lax.dot_general(
        x, rw, (((1,), (1,)), ((), ())), preferred_element_type=jnp.float32)
    logits_ref[...] = logits
    lane = jax.lax.broadcasted_iota(jnp.int32, logits.shape, 1)
    m1 = jnp.max(logits, axis=1, keepdims=True)
    i1 = jnp.min(jnp.where(logits == m1, lane, E), axis=1, keepdims=True)
    l2 = jnp.where(lane == i1, _NEG, logits)
    m2 = jnp.max(l2, axis=1, keepdims=True)
    i2 = jnp.min(jnp.where(l2 == m2, lane, E), axis=1, keepdims=True)
    p2 = jnp.exp(m2 - m1)
    s = 1.0 + p2
    w1_ref[...] = 1.0 / s
    w2_ref[...] = p2 / s
    i1_ref[...] = i1
    i2_ref[...] = i2


def _router(router_input, router_w):
    return pl.pallas_call(
        _router_body,
        out_shape=[
            jax.ShapeDtypeStruct((T, E), jnp.float32),
            jax.ShapeDtypeStruct((T, 1), jnp.int32),
            jax.ShapeDtypeStruct((T, 1), jnp.int32),
            jax.ShapeDtypeStruct((T, 1), jnp.float32),
            jax.ShapeDtypeStruct((T, 1), jnp.float32),
        ],
    )(router_input, router_w)


RB = 512
NRB = TK // RB


def _rank_body(e_ref, rank_ref, counts_ref):
    g = pl.program_id(0)

    @pl.when(g == 0)
    def _():
        counts_ref[...] = jnp.zeros_like(counts_ref)

    e = e_ref[...]
    lane = jax.lax.broadcasted_iota(jnp.int32, (RB, E), 1)
    oh = (e == lane)
    ohf = oh.astype(jnp.float32)
    r = jax.lax.broadcasted_iota(jnp.int32, (RB, RB), 0)
    c = jax.lax.broadcasted_iota(jnp.int32, (RB, RB), 1)
    tri = (r > c).astype(jnp.float32)
    within = jax.lax.dot_general(
        tri, ohf, (((1,), (0,)), ((), ())),
        preferred_element_type=jnp.float32)
    base = counts_ref[...]
    pos_all = base + within.astype(jnp.int32)
    rank_ref[...] = jnp.sum(jnp.where(oh, pos_all, 0), axis=1, keepdims=True)
    counts_ref[...] = base + jnp.sum(ohf, axis=0,
                                     keepdims=True).astype(jnp.int32)


def _rank(e2):
    return pl.pallas_call(
        _rank_body,
        grid=(NRB,),
        in_specs=[pl.BlockSpec((RB, 1), lambda g: (g, 0))],
        out_specs=[pl.BlockSpec((RB, 1), lambda g: (g, 0)),
                   pl.BlockSpec((1, E), lambda g: (0, 0))],
        out_shape=[jax.ShapeDtypeStruct((TK, 1), jnp.int32),
                   jax.ShapeDtypeStruct((1, E), jnp.int32)],
        compiler_params=pltpu.CompilerParams(
            dimension_semantics=("arbitrary",)),
    )(e2)


def _ffn_body(be_ref, na_ref, x_ref, wg_ref, wu_ref, wd_ref, wrow_ref, y_ref):
    b = pl.program_id(0)
    y_ref[...] = jnp.zeros_like(y_ref)

    @pl.when(b < na_ref[0])
    def _():
        x = x_ref[...].astype(jnp.bfloat16)
        g = jnp.maximum(
            jnp.dot(x, wg_ref[0].astype(jnp.bfloat16),
                    preferred_element_type=jnp.float32), 0.0)
        u = jnp.dot(x, wu_ref[0].astype(jnp.bfloat16),
                    preferred_element_type=jnp.float32)
        h = (g * u).astype(jnp.bfloat16)
        y_ref[...] = jnp.dot(h, wd_ref[0].astype(jnp.bfloat16),
                             preferred_element_type=jnp.float32) * wrow_ref[...]


def _grouped_ffn(be, na, xs, w_gate, w_up, w_down, wrow):
    grid_spec = pltpu.PrefetchScalarGridSpec(
        num_scalar_prefetch=2,
        grid=(NB,),
        in_specs=[
            pl.BlockSpec((BT, D), lambda b, be, na: (b, 0)),
            pl.BlockSpec((1, D, F), lambda b, be, na: (be[b], 0, 0)),
            pl.BlockSpec((1, D, F), lambda b, be, na: (be[b], 0, 0)),
            pl.BlockSpec((1, F, D), lambda b, be, na: (be[b], 0, 0)),
            pl.BlockSpec((BT, 1), lambda b, be, na: (b, 0)),
        ],
        out_specs=pl.BlockSpec((BT, D), lambda b, be, na: (b, 0)),
    )
    return pl.pallas_call(
        _ffn_body,
        grid_spec=grid_spec,
        out_shape=jax.ShapeDtypeStruct((NPAD, D), jnp.float32),
        compiler_params=pltpu.CompilerParams(
            dimension_semantics=("arbitrary",)),
    )(be, na, xs, w_gate, w_up, w_down, wrow)


def kernel(router_input, hidden_states, router_w, w_gate, w_up, w_down):
    logits, i1, i2, w1, w2 = _router(router_input, router_w)
    e_flat = jnp.concatenate([i1, i2], axis=0)
    w_flat = jnp.concatenate([w1, w2], axis=0)[:, 0]

    rank2, counts2 = _rank(e_flat)
    counts = counts2[0]
    pc = ((counts + BT - 1) // BT) * BT
    ends = jnp.cumsum(pc)
    start = ends - pc
    pos = start[e_flat[:, 0]] + rank2[:, 0]
    perm = jnp.zeros((NPAD,), jnp.int32).at[pos].set(
        jnp.arange(TK, dtype=jnp.int32) % T)
    wrow = jnp.zeros((NPAD,), jnp.float32).at[pos].set(w_flat)
    total = ends[-1]
    na = (total // BT).astype(jnp.int32)
    be = jnp.sum(
        (jnp.arange(NB, dtype=jnp.int32)[:, None] * BT) >= ends[None, :],
        axis=1).astype(jnp.int32)
    be_last = jnp.take(be, na - 1)
    be = jnp.where(jnp.arange(NB) < na, be, be_last).astype(jnp.int32)

    xs = jnp.take(hidden_states, perm, axis=0)
    y = _grouped_ffn(be, na[None], xs, w_gate, w_up, w_down, wrow[:, None])
    out = (jnp.take(y, pos[:T], axis=0)
           + jnp.take(y, pos[T:], axis=0))
    return out, logits


# speedup vs baseline: 1.0699x; 1.0699x over previous
import jax
import jax.numpy as jnp
from jax.experimental import pallas as pl
from jax.experimental.pallas import tpu as pltpu

T = 2048
D = 1024
F = 2048
E = 8
TOPK = 2
TK = T * TOPK
BT = 128
NPAD = TK + E * BT
NB = NPAD // BT
NBP = 48

_NEG = -1e30


def _router_body(x_ref, rw_ref, logits_ref, p1_ref, p2_ref, w1_ref, w2_ref,
                 be_ref, na_ref):
    x = x_ref[...]
    rw = rw_ref[...]
    logits = jax.lax.dot_general(
        x, rw, (((1,), (1,)), ((), ())), preferred_element_type=jnp.float32)
    logits_ref[...] = logits

    lane = jax.lax.broadcasted_iota(jnp.int32, (T, E), 1)
    m1 = jnp.max(logits, axis=1, keepdims=True)
    i1 = jnp.min(jnp.where(logits == m1, lane, E), axis=1, keepdims=True)
    l2 = jnp.where(lane == i1, _NEG, logits)
    m2 = jnp.max(l2, axis=1, keepdims=True)
    i2 = jnp.min(jnp.where(l2 == m2, lane, E), axis=1, keepdims=True)
    p2v = jnp.exp(m2 - m1)
    s = 1.0 + p2v
    w1_ref[...] = 1.0 / s
    w2_ref[...] = p2v / s

    oh1 = lane == i1
    oh2 = lane == i2
    ohf = jnp.concatenate([oh1.astype(jnp.float32),
                           oh2.astype(jnp.float32)], axis=1)
    r = jax.lax.broadcasted_iota(jnp.int32, (T, T), 0)
    c = jax.lax.broadcasted_iota(jnp.int32, (T, T), 1)
    tri = (r > c).astype(jnp.float32)
    cums = jax.lax.dot_general(
        tri, ohf, (((1,), (0,)), ((), ())),
        preferred_element_type=jnp.float32)
    tots = jnp.sum(ohf, axis=0, keepdims=True)
    counts = (tots[:, :E] + tots[:, E:]).astype(jnp.int32)
    pc = ((counts + BT - 1) // BT) * BT
    le8r = jax.lax.broadcasted_iota(jnp.int32, (E, E), 0)
    le8c = jax.lax.broadcasted_iota(jnp.int32, (E, E), 1)
    le8 = (le8r <= le8c).astype(jnp.float32)
    ends = jax.lax.dot_general(
        pc.astype(jnp.float32), le8, (((1,), (0,)), ((), ())),
        preferred_element_type=jnp.float32).astype(jnp.int32)
    start = ends - pc
    rank1 = jnp.sum(jnp.where(oh1, cums[:, :E], 0.0), axis=1,
                    keepdims=True).astype(jnp.int32)
    rank2 = jnp.sum(jnp.where(oh2, cums[:, E:] + tots[:, :E], 0.0), axis=1,
                    keepdims=True).astype(jnp.int32)
    st1 = jnp.sum(jnp.where(oh1, start, 0), axis=1, keepdims=True)
    st2 = jnp.sum(jnp.where(oh2, start, 0), axis=1, keepdims=True)
    p1_ref[...] = st1 + rank1
    p2_ref[...] = st2 + rank2

    total = jnp.max(ends)
    na = total // BT
    bv = jax.lax.broadcasted_iota(jnp.int32, (NBP, 1), 0)
    berow = jnp.sum((bv * BT >= ends).astype(jnp.int32), axis=1,
                    keepdims=True)
    be_last = jnp.sum(((na - 1) * BT >= ends).astype(jnp.int32))
    be_ref[...] = jnp.where(bv < na, berow, be_last)
    na_ref[...] = jnp.full((1, 1), na, jnp.int32)


def _router(router_input, router_w):
    return pl.pallas_call(
        _router_body,
        out_shape=[
            jax.ShapeDtypeStruct((T, E), jnp.float32),
            jax.ShapeDtypeStruct((T, 1), jnp.int32),
            jax.ShapeDtypeStruct((T, 1), jnp.int32),
            jax.ShapeDtypeStruct((T, 1), jnp.float32),
            jax.ShapeDtypeStruct((T, 1), jnp.float32),
            jax.ShapeDtypeStruct((NBP, 1), jnp.int32),
            jax.ShapeDtypeStruct((1, 1), jnp.int32),
        ],
    )(router_input, router_w)


def _ffn_body(be_ref, na_ref, x_ref, wg_ref, wu_ref, wd_ref, wrow_ref, y_ref):
    b = pl.program_id(0)
    y_ref[...] = jnp.zeros_like(y_ref)

    @pl.when(b < na_ref[0])
    def _():
        x = x_ref[...]
        g = jnp.maximum(
            jnp.dot(x, wg_ref[0], preferred_element_type=jnp.float32), 0.0)
        u = jnp.dot(x, wu_ref[0], preferred_element_type=jnp.float32)
        y_ref[...] = jnp.dot(g * u, wd_ref[0],
                             preferred_element_type=jnp.float32) * wrow_ref[...]


def _grouped_ffn(be, na, xs, w_gate, w_up, w_down, wrow):
    grid_spec = pltpu.PrefetchScalarGridSpec(
        num_scalar_prefetch=2,
        grid=(NB,),
        in_specs=[
            pl.BlockSpec((BT, D), lambda b, be, na: (b, 0)),
            pl.BlockSpec((1, D, F), lambda b, be, na: (be[b], 0, 0)),
            pl.BlockSpec((1, D, F), lambda b, be, na: (be[b], 0, 0)),
            pl.BlockSpec((1, F, D), lambda b, be, na: (be[b], 0, 0)),
            pl.BlockSpec((BT, 1), lambda b, be, na: (b, 0)),
        ],
        out_specs=pl.BlockSpec((BT, D), lambda b, be, na: (b, 0)),
    )
    return pl.pallas_call(
        _ffn_body,
        grid_spec=grid_spec,
        out_shape=jax.ShapeDtypeStruct((NPAD, D), jnp.float32),
        compiler_params=pltpu.CompilerParams(
            dimension_semantics=("arbitrary",)),
    )(be, na, xs, w_gate, w_up, w_down, wrow)


def kernel(router_input, hidden_states, router_w, w_gate, w_up, w_down):
    logits, p1, p2, w1, w2, be, na = _router(router_input, router_w)
    p1c, p2c = p1[:, 0], p2[:, 0]

    tok = jnp.arange(T, dtype=jnp.int32)
    perm = (jnp.zeros((NPAD,), jnp.int32).at[p1c].set(tok).at[p2c].set(tok))
    wrow = (jnp.zeros((NPAD,), jnp.float32)
            .at[p1c].set(w1[:, 0]).at[p2c].set(w2[:, 0]))
    xs = jnp.take(hidden_states, perm, axis=0)

    y = _grouped_ffn(be[:, 0], na[0], xs, w_gate, w_up, w_down, wrow[:, None])

    out = jnp.take(y, p1c, axis=0) + jnp.take(y, p2c, axis=0)
    return out, logits


# speedup vs baseline: 1.1436x; 1.0688x over previous
import jax
import jax.numpy as jnp
from jax.experimental import pallas as pl
from jax.experimental.pallas import tpu as pltpu

T = 2048
D = 1024
F = 2048
E = 8
TOPK = 2
TK = T * TOPK
BT = 128
NPAD = TK + E * BT
NB = NPAD // BT
NBP = 48

_NEG = -1e30


def _router_body(x_ref, rw_ref, logits_ref, p1_ref, p2_ref, w1_ref, w2_ref,
                 be_ref, na_ref):
    x = x_ref[...]
    rw = rw_ref[...]
    logits = jax.lax.dot_general(
        x, rw, (((1,), (1,)), ((), ())), preferred_element_type=jnp.float32)
    logits_ref[...] = logits

    lane = jax.lax.broadcasted_iota(jnp.int32, (T, E), 1)
    m1 = jnp.max(logits, axis=1, keepdims=True)
    i1 = jnp.min(jnp.where(logits == m1, lane, E), axis=1, keepdims=True)
    l2 = jnp.where(lane == i1, _NEG, logits)
    m2 = jnp.max(l2, axis=1, keepdims=True)
    i2 = jnp.min(jnp.where(l2 == m2, lane, E), axis=1, keepdims=True)
    p2v = jnp.exp(m2 - m1)
    s = 1.0 + p2v
    w1_ref[...] = 1.0 / s
    w2_ref[...] = p2v / s

    oh1 = lane == i1
    oh2 = lane == i2
    ohf = jnp.concatenate([oh1.astype(jnp.float32),
                           oh2.astype(jnp.float32)], axis=1)
    r = jax.lax.broadcasted_iota(jnp.int32, (T, T), 0)
    c = jax.lax.broadcasted_iota(jnp.int32, (T, T), 1)
    tri = (r > c).astype(jnp.float32)
    cums = jax.lax.dot_general(
        tri, ohf, (((1,), (0,)), ((), ())),
        preferred_element_type=jnp.float32)
    tots = jnp.sum(ohf, axis=0, keepdims=True)
    counts = (tots[:, :E] + tots[:, E:]).astype(jnp.int32)
    pc = ((counts + BT - 1) // BT) * BT
    le8r = jax.lax.broadcasted_iota(jnp.int32, (E, E), 0)
    le8c = jax.lax.broadcasted_iota(jnp.int32, (E, E), 1)
    le8 = (le8r <= le8c).astype(jnp.float32)
    ends = jax.lax.dot_general(
        pc.astype(jnp.float32), le8, (((1,), (0,)), ((), ())),
        preferred_element_type=jnp.float32).astype(jnp.int32)
    start = ends - pc
    rank1 = jnp.sum(jnp.where(oh1, cums[:, :E], 0.0), axis=1,
                    keepdims=True).astype(jnp.int32)
    rank2 = jnp.sum(jnp.where(oh2, cums[:, E:] + tots[:, :E], 0.0), axis=1,
                    keepdims=True).astype(jnp.int32)
    st1 = jnp.sum(jnp.where(oh1, start, 0), axis=1, keepdims=True)
    st2 = jnp.sum(jnp.where(oh2, start, 0), axis=1, keepdims=True)
    p1_ref[...] = st1 + rank1
    p2_ref[...] = st2 + rank2

    total = jnp.max(ends)
    na = total // BT
    bv = jax.lax.broadcasted_iota(jnp.int32, (NBP, 1), 0)
    berow = jnp.sum((bv * BT >= ends).astype(jnp.int32), axis=1,
                    keepdims=True)
    be_last = jnp.sum(((na - 1) * BT >= ends).astype(jnp.int32))
    be_ref[...] = jnp.where(bv < na, berow, be_last)
    na_ref[...] = jnp.full((1, 1), na, jnp.int32)


def _router(router_input, router_w):
    return pl.pallas_call(
        _router_body,
        out_shape=[
            jax.ShapeDtypeStruct((T, E), jnp.float32),
            jax.ShapeDtypeStruct((T, 1), jnp.int32),
            jax.ShapeDtypeStruct((T, 1), jnp.int32),
            jax.ShapeDtypeStruct((T, 1), jnp.float32),
            jax.ShapeDtypeStruct((T, 1), jnp.float32),
            jax.ShapeDtypeStruct((NBP, 1), jnp.int32),
            jax.ShapeDtypeStruct((1, 1), jnp.int32),
        ],
    )(router_input, router_w)


def _ffn_body(be_ref, na_ref, x_ref, wg_ref, wu_ref, wd_ref, wrow_ref, y_ref):
    b = pl.program_id(0)
    y_ref[...] = jnp.zeros_like(y_ref)

    @pl.when(b < na_ref[0])
    def _():
        x = x_ref[...]
        g = jnp.maximum(
            jnp.dot(x, wg_ref[0], preferred_element_type=jnp.float32), 0.0)
        u = jnp.dot(x, wu_ref[0], preferred_element_type=jnp.float32)
        y_ref[...] = jnp.dot(g * u, wd_ref[0],
                             preferred_element_type=jnp.float32) * wrow_ref[...]


def _grouped_ffn(be, na, xs, w_gate, w_up, w_down, wrow):
    grid_spec = pltpu.PrefetchScalarGridSpec(
        num_scalar_prefetch=2,
        grid=(NB,),
        in_specs=[
            pl.BlockSpec((BT, D), lambda b, be, na: (b, 0)),
            pl.BlockSpec((1, D, F), lambda b, be, na: (be[b], 0, 0)),
            pl.BlockSpec((1, D, F), lambda b, be, na: (be[b], 0, 0)),
            pl.BlockSpec((1, F, D), lambda b, be, na: (be[b], 0, 0)),
            pl.BlockSpec((BT, 1), lambda b, be, na: (b, 0)),
        ],
        out_specs=pl.BlockSpec((BT, D), lambda b, be, na: (b, 0)),
    )
    return pl.pallas_call(
        _ffn_body,
        grid_spec=grid_spec,
        out_shape=jax.ShapeDtypeStruct((NPAD, D), jnp.float32),
        compiler_params=pltpu.CompilerParams(
            dimension_semantics=("arbitrary",)),
    )(be, na, xs, w_gate, w_up, w_down, wrow)


def kernel(router_input, hidden_states, router_w, w_gate, w_up, w_down):
    logits, p1, p2, w1, w2, be, na = _router(router_input, router_w)
    p1c, p2c = p1[:, 0], p2[:, 0]

    wrow = (jnp.zeros((NPAD,), jnp.float32)
            .at[p1c].set(w1[:, 0]).at[p2c].set(w2[:, 0]))
    xs = (jnp.zeros((NPAD, D), jnp.float32)
          .at[p1c].set(hidden_states).at[p2c].set(hidden_states))

    y = _grouped_ffn(be[:, 0], na[0], xs, w_gate, w_up, w_down, wrow[:, None])

    out = jnp.take(y, p1c, axis=0) + jnp.take(y, p2c, axis=0)
    return out, logits
